# baseline (speedup 1.0000x reference)
import jax
import jax.numpy as jnp
from jax import lax
from jax.experimental import pallas as pl
from jax.experimental.pallas import tpu as pltpu
from jax.experimental.pallas import tpu_sc as plsc

D_MODEL = 64
SCALE = 8.0

NC = 2
NS = 16
NW = NC * NS

GRP = 128
K = 5
CHUNK = GRP * K
LANES = 16


def _body(x_hbm, table_hbm, out_hbm, idx_v, rows_v, gsem):
    b_per_w = idx_v.shape[0]
    nchunks = b_per_w // CHUNK
    wid = lax.axis_index("s") * NC + lax.axis_index("c")
    base = pl.multiple_of(wid * b_per_w, 8)

    pltpu.sync_copy(x_hbm.at[pl.ds(base, b_per_w)], idx_v)

    def chunk_body(c, carry):
        off = pl.multiple_of(c * CHUNK, CHUNK)
        handles = []
        for k in range(K):
            handles.append(pltpu.async_copy(
                table_hbm.at[idx_v.at[pl.ds(off + k * GRP, GRP)]],
                rows_v.at[pl.ds(k * GRP, GRP)],
                gsem,
            ))
        for h in handles:
            h.wait()

        def scale_row(r, carry2):
            for j in range(D_MODEL // LANES):
                sl = (r, pl.ds(j * LANES, LANES))
                rows_v[sl] = rows_v[sl] * SCALE
            return carry2
        lax.fori_loop(0, CHUNK, scale_row, 0, unroll=2)

        pltpu.sync_copy(rows_v, out_hbm.at[pl.ds(base + off, CHUNK)])
        return carry
    lax.fori_loop(0, nchunks, chunk_body, 0)


def kernel(x, table):
    b_total = x.size
    b_per_w = b_total // NW
    xf = x.reshape(-1).astype(jnp.int32)
    mesh = plsc.VectorSubcoreMesh(core_axis_name="c", subcore_axis_name="s")
    out = pl.kernel(
        _body,
        mesh=mesh,
        compiler_params=pltpu.CompilerParams(use_tc_tiling_on_sc=False),
        out_type=jax.ShapeDtypeStruct((b_total, D_MODEL), jnp.float32),
        scratch_types=[
            pltpu.VMEM((b_per_w,), jnp.int32),
            pltpu.VMEM((CHUNK, D_MODEL), jnp.float32),
            pltpu.SemaphoreType.DMA,
        ],
    )(xf, table)
    return out.reshape(*x.shape, D_MODEL)


# speedup vs baseline: 1.0667x; 1.0667x over previous
import jax
import jax.numpy as jnp
from jax import lax
from jax.experimental import pallas as pl
from jax.experimental.pallas import tpu as pltpu
from jax.experimental.pallas import tpu_sc as plsc

D_MODEL = 64
SCALE = 8.0

NC = 2
NS = 16
NW = NC * NS

GRP = 128
K = 5
CHUNK = GRP * K
LANES = 16


def _body(x_hbm, table_hbm, out_hbm, idx_v, rows0, rows1, gsem, ssem):
    ngrp = idx_v.shape[0]
    b_per_w = ngrp * GRP
    nchunks = b_per_w // CHUNK
    wid = lax.axis_index("s") * NC + lax.axis_index("c")
    base = pl.multiple_of(wid * b_per_w, 8)

    pltpu.sync_copy(x_hbm.at[pl.ds(wid * ngrp, ngrp)], idx_v)

    def gather_copies(c, buf):
        return [
            pltpu.make_async_copy(
                table_hbm.at[idx_v.at[c * K + k]],
                buf.at[pl.ds(k * GRP, GRP)],
                gsem,
            )
            for k in range(K)
        ]

    def scatter_copy(c, buf):
        return pltpu.make_async_copy(
            buf, out_hbm.at[pl.ds(base + c * CHUNK, CHUNK)], ssem)

    def scale(buf):
        def scale_row(r, carry):
            for j in range(D_MODEL // LANES):
                sl = (r, pl.ds(j * LANES, LANES))
                buf[sl] = buf[sl] * SCALE
            return carry
        lax.fori_loop(0, CHUNK, scale_row, 0, unroll=4)

    for cp in gather_copies(0, rows0):
        cp.start()

    def step(g, carry):
        for b, (buf, obuf) in enumerate(((rows0, rows1), (rows1, rows0))):
            c = 2 * g + b
            for cp in gather_copies(c, buf):
                cp.wait()

            @pl.when(c >= 1)
            def _():
                scatter_copy(c - 1, obuf).wait()

            @pl.when(c + 1 < nchunks)
            def _():
                for cp in gather_copies(c + 1, obuf):
                    cp.start()

            scale(buf)
            scatter_copy(c, buf).start()
        return carry

    lax.fori_loop(0, nchunks // 2, step, 0)
    scatter_copy(nchunks - 1, rows1).wait()


def kernel(x, table):
    b_total = x.size
    ngrp = b_total // (NW * GRP)
    xg = x.reshape(-1, GRP).astype(jnp.int32)
    mesh = plsc.VectorSubcoreMesh(core_axis_name="c", subcore_axis_name="s")
    out = pl.kernel(
        _body,
        mesh=mesh,
        compiler_params=pltpu.CompilerParams(use_tc_tiling_on_sc=False),
        out_type=jax.ShapeDtypeStruct((b_total, D_MODEL), jnp.float32),
        scratch_types=[
            pltpu.VMEM((ngrp, GRP), jnp.int32),
            pltpu.VMEM((CHUNK, D_MODEL), jnp.float32),
            pltpu.VMEM((CHUNK, D_MODEL), jnp.float32),
            pltpu.SemaphoreType.DMA,
            pltpu.SemaphoreType.DMA,
        ],
    )(xg, table)
    return out.reshape(*x.shape, D_MODEL)
